# Initial kernel scaffold; baseline (speedup 1.0000x reference)
import jax
import jax.numpy as jnp
from jax.experimental import pallas as pl


def kernel(output, mask, ind, target):
    raise NotImplementedError("write your pallas kernel here")


# speedup vs baseline: 1.0489x; 1.0489x over previous
import functools

import jax
import jax.numpy as jnp
from jax import lax
from jax.experimental import pallas as pl
from jax.experimental.pallas import tpu as pltpu
from jax.experimental.pallas import tpu_sc as plsc

_L = 16
_G = 128


def _make_sc_kernel(B, C, HW, KP):
    ng = (C * KP) // _G
    gpc = KP // _G
    mesh = plsc.VectorSubcoreMesh(core_axis_name="core", subcore_axis_name="sub")

    @functools.partial(
        pl.kernel,
        mesh=mesh,
        out_type=jax.ShapeDtypeStruct((B, _L), jnp.float32),
        scratch_types=[
            pltpu.VMEM((ng, _G), jnp.int32),
            pltpu.VMEM((C * KP,), jnp.float32),
            pltpu.VMEM((KP,), jnp.float32),
            pltpu.VMEM((_G,), jnp.float32),
            pltpu.VMEM((_L,), jnp.float32),
            pltpu.SemaphoreType.DMA,
        ],
    )
    def sc_kernel(outp_hbm, idx_hbm, mkf_hbm, tgt_hbm, out_hbm,
                  idx_v, tgt_v, mkf_v, pred_v, acc_v, sem):
        ncore = lax.axis_size("core")
        b = lax.axis_index("sub") * ncore + lax.axis_index("core")

        pltpu.sync_copy(idx_hbm.at[b], idx_v)
        pltpu.sync_copy(mkf_hbm.at[b], mkf_v)
        pltpu.sync_copy(tgt_hbm.at[b], tgt_v)

        def body_g(g, acc):
            pltpu.async_copy(outp_hbm.at[idx_v.at[g]], pred_v, sem).wait()
            koff = lax.rem(g, gpc) * _G
            for j in range(_G // _L):
                mk = mkf_v[pl.ds(koff + j * _L, _L)]
                tg = tgt_v[pl.ds(g * _G + j * _L, _L)]
                pr = pred_v[pl.ds(j * _L, _L)]
                acc = acc + mk * jnp.abs(pr - tg)
            return acc

        acc = lax.fori_loop(0, ng, body_g, jnp.zeros((_L,), jnp.float32))
        acc_v[...] = acc
        pltpu.sync_copy(acc_v, out_hbm.at[b])

    return sc_kernel


def kernel(output, mask, ind, target):
    B, C, H, W = output.shape
    K = ind.shape[1]
    HW = H * W
    KP = ((K + _G - 1) // _G) * _G

    outp = output.reshape(B * C * HW)
    ind_pad = jnp.zeros((B, KP), jnp.int32).at[:, :K].set(ind.astype(jnp.int32))
    base = (jnp.arange(B, dtype=jnp.int32)[:, None] * C
            + jnp.arange(C, dtype=jnp.int32)[None, :]) * HW
    abs_idx = (base[:, :, None] + ind_pad[:, None, :]).reshape(B, (C * KP) // _G, _G)
    mkf = jnp.zeros((B, KP), jnp.float32).at[:, :K].set(mask.astype(jnp.float32))
    tgt = (
        jnp.zeros((B, C, KP), jnp.float32)
        .at[:, :, :K].set(jnp.transpose(target, (0, 2, 1)))
        .reshape(B, C * KP)
    )

    partial = _make_sc_kernel(B, C, HW, KP)(outp, abs_idx, mkf, tgt)
    return jnp.sum(partial) / (B * K * C)


# speedup vs baseline: 3.1566x; 3.0095x over previous
import functools

import jax
import jax.numpy as jnp
from jax import lax
from jax.experimental import pallas as pl
from jax.experimental.pallas import tpu as pltpu
from jax.experimental.pallas import tpu_sc as plsc

_L = 16
_G = 128


_NB = 8


def _make_sc_kernel(B, C, HW, KP):
    ng = (C * KP) // _G
    gpc = KP // _G
    nblk = ng // _NB
    mesh = plsc.VectorSubcoreMesh(core_axis_name="core", subcore_axis_name="sub")

    @functools.partial(
        pl.kernel,
        mesh=mesh,
        out_type=jax.ShapeDtypeStruct((B, _L), jnp.float32),
        scratch_types=[
            pltpu.VMEM((ng, _G), jnp.int32),
            pltpu.VMEM((C * KP,), jnp.float32),
            pltpu.VMEM((KP,), jnp.float32),
            pltpu.VMEM((2, _NB, _G), jnp.float32),
            pltpu.VMEM((_L,), jnp.float32),
            pltpu.SemaphoreType.DMA,
            pltpu.SemaphoreType.DMA,
        ],
    )
    def sc_kernel(outp_hbm, idx_hbm, mkf_hbm, tgt_hbm, out_hbm,
                  idx_v, tgt_v, mkf_v, pred_v, acc_v, sem0, sem1):
        ncore = lax.axis_size("core")
        b = lax.axis_index("sub") * ncore + lax.axis_index("core")

        pltpu.sync_copy(idx_hbm.at[b], idx_v)
        pltpu.sync_copy(mkf_hbm.at[b], mkf_v)
        pltpu.sync_copy(tgt_hbm.at[b], tgt_v)

        def fire(blk, buf, sem):
            for s in range(_NB):
                pltpu.async_copy(
                    outp_hbm.at[idx_v.at[blk * _NB + s]], pred_v.at[buf, s], sem)

        def consume(blk, buf, sem, acc):
            for s in range(_NB):
                g = blk * _NB + s
                pltpu.make_async_copy(
                    outp_hbm.at[idx_v.at[g]], pred_v.at[buf, s], sem).wait()
                koff = (s % gpc) * _G
                for j in range(_G // _L):
                    mk = mkf_v[pl.ds(koff + j * _L, _L)]
                    tg = tgt_v[pl.ds(g * _G + j * _L, _L)]
                    pr = pred_v[buf, s, pl.ds(j * _L, _L)]
                    acc = acc + mk * jnp.abs(pr - tg)
            return acc

        fire(0, 0, sem0)

        def body(i, acc):
            blk0 = i * 2
            fire(blk0 + 1, 1, sem1)
            acc = consume(blk0, 0, sem0, acc)

            @pl.when(blk0 + 2 < nblk)
            def _():
                fire(blk0 + 2, 0, sem0)

            return consume(blk0 + 1, 1, sem1, acc)

        acc = lax.fori_loop(0, nblk // 2, body, jnp.zeros((_L,), jnp.float32))
        acc_v[...] = acc
        pltpu.sync_copy(acc_v, out_hbm.at[b])

    return sc_kernel


def kernel(output, mask, ind, target):
    B, C, H, W = output.shape
    K = ind.shape[1]
    HW = H * W
    KP = ((K + _G - 1) // _G) * _G

    outp = output.reshape(B * C * HW)
    ind_pad = jnp.zeros((B, KP), jnp.int32).at[:, :K].set(ind.astype(jnp.int32))
    base = (jnp.arange(B, dtype=jnp.int32)[:, None] * C
            + jnp.arange(C, dtype=jnp.int32)[None, :]) * HW
    abs_idx = (base[:, :, None] + ind_pad[:, None, :]).reshape(B, (C * KP) // _G, _G)
    mkf = jnp.zeros((B, KP), jnp.float32).at[:, :K].set(mask.astype(jnp.float32))
    tgt = (
        jnp.zeros((B, C, KP), jnp.float32)
        .at[:, :, :K].set(jnp.transpose(target, (0, 2, 1)))
        .reshape(B, C * KP)
    )

    partial = _make_sc_kernel(B, C, HW, KP)(outp, abs_idx, mkf, tgt)
    return jnp.sum(partial) / (B * K * C)
